# baseline (speedup 1.0000x reference)
import jax
import jax.numpy as jnp
from jax import lax
from jax.experimental import pallas as pl
from jax.experimental.pallas import tpu as pltpu
from jax.experimental.pallas import tpu_sc as plsc

N_NODES = 50000
D = 512
NSEG = 256
L = 16
NC = 2
NS = 16
NW = NC * NS
SPW = NSEG // NW

GRAN = 16
NGRP = 3136
TCB = 1024
NPAD = NGRP * GRAN

GR = 16
WR = 24
QUART = 4
CQ = D // QUART // L
HALF = 2
CV = D // HALF // L


def _tc_group_body(feat_ref, mask_ref, out_ref):
    f3 = feat_ref[...].reshape(TCB // GRAN, GRAN, D)
    mb = mask_ref[0][:, :, None]
    out_ref[:, :D] = jnp.sum(f3 * mb, axis=1)
    out_ref[:, D:] = jnp.max(jnp.where(mb > 0.5, f3, -jnp.inf), axis=1)


def _tc_group_partials(feat, segment_ids):
    ids_p = jnp.pad(segment_ids, (0, NPAD - N_NODES), constant_values=NSEG)
    ids2d = ids_p.reshape(NGRP, GRAN)
    mask3 = (ids2d == ids2d[:, :1]).astype(jnp.float32).reshape(
        NPAD // TCB, TCB // GRAN, GRAN)
    return pl.pallas_call(
        _tc_group_body,
        grid=(NPAD // TCB,),
        in_specs=[
            pl.BlockSpec((TCB, D), lambda i: (i, 0)),
            pl.BlockSpec((1, TCB // GRAN, GRAN), lambda i: (i, 0, 0)),
        ],
        out_specs=pl.BlockSpec((TCB // GRAN, 2 * D), lambda i: (i, 0)),
        out_shape=jax.ShapeDtypeStruct((NGRP, 2 * D), jnp.float32),
    )(feat, mask3)


def _comb_body(lo_hbm, feat_hbm, ids_hbm, out_hbm,
               idsv, gbuf, wbuf, ostage, gsem0, gsem1, wsem0, wsem1):
    wid = lax.axis_index("s") * NC + lax.axis_index("c")
    sbase = wid * SPW

    pltpu.sync_copy(ids_hbm, idsv)

    targets = sbase + lax.iota(jnp.int32, L)
    pos = jnp.zeros((L,), jnp.int32)
    step = 32768
    while step >= 1:
        npos = pos + step
        idx = jnp.minimum(npos - 1, N_NODES - 1)
        vals = plsc.load_gather(idsv, [idx])
        ok = (npos <= N_NODES) & (vals < targets)
        pos = jnp.where(ok, npos, pos)
        step //= 2

    s_bnds = [pos[k] for k in range(SPW + 1)]
    glo = [(s_bnds[k] + GRAN - 1) // GRAN for k in range(SPW + 1)]

    zeros = jnp.zeros((L,), jnp.float32)
    ninf = jnp.full((L,), -jnp.inf, jnp.float32)

    def init_body(k, c):
        for j in range(D // L):
            ostage[k, pl.ds(j * L, L)] = zeros
            ostage[k, pl.ds(D + j * L, L)] = ninf
        return c

    lax.fori_loop(0, SPW, init_body, 0)

    def wdesc(k, b):
        ws = jnp.minimum((s_bnds[k] // 8) * 8, N_NODES - WR)
        sem = wsem0 if b == 0 else wsem1
        return pltpu.make_async_copy(
            feat_hbm.at[pl.ds(ws, WR)], wbuf.at[b], sem)

    wdesc(0, 0).start()
    wdesc(1, 1).start()

    G0 = glo[0]
    G8 = glo[SPW]
    gg_lo = G0 // GR
    gnb = jnp.where(G8 > G0, (G8 + GR - 1) // GR - gg_lo, 0)

    def gb_start(t):
        return jnp.minimum((gg_lo + t) * GR, NGRP - GR)

    def gdesc(t, b):
        sem = gsem0 if b == 0 else gsem1
        return pltpu.make_async_copy(
            lo_hbm.at[pl.ds(gb_start(t), GR)], gbuf.at[b], sem)

    @pl.when(gnb > 0)
    def _():
        gdesc(0, 0).start()

    @pl.when(gnb > 1)
    def _():
        gdesc(1, 1).start()

    def process_g(t, b):
        g = (gg_lo + t) * GR
        p_lo = jnp.maximum(G0, g)
        p_hi = jnp.minimum(G8, g + GR)
        bstart = gb_start(t)
        buf = gbuf.at[b]

        for k in range(SPW):
            a = jnp.maximum(glo[k], p_lo)
            e = jnp.minimum(glo[k + 1], p_hi)

            @pl.when(e > a)
            def _():
                for h in range(QUART):
                    scol = h * (D // QUART)
                    mcol = D + h * (D // QUART)
                    carry0 = tuple(
                        ostage[k, pl.ds(scol + j * L, L)] for j in range(CQ)
                    ) + tuple(
                        ostage[k, pl.ds(mcol + j * L, L)] for j in range(CQ)
                    )

                    def row_body(r, carry):
                        ro = r - bstart
                        ss = [buf[ro, pl.ds(scol + j * L, L)]
                              for j in range(CQ)]
                        ms = [buf[ro, pl.ds(mcol + j * L, L)]
                              for j in range(CQ)]
                        sums = tuple(s + f for s, f in zip(carry[:CQ], ss))
                        maxs = tuple(jnp.maximum(m, f)
                                     for m, f in zip(carry[CQ:], ms))
                        return sums + maxs

                    carry = lax.fori_loop(a, e, row_body, carry0)
                    for j in range(CQ):
                        ostage[k, pl.ds(scol + j * L, L)] = carry[j]
                        ostage[k, pl.ds(mcol + j * L, L)] = carry[CQ + j]

    def pair_body(u, c):
        for b in range(2):
            t = u * 2 + b

            @pl.when(t < gnb)
            def _():
                gdesc(t, b).wait()
                process_g(t, b)

                @pl.when(t + 2 < gnb)
                def _():
                    gdesc(t + 2, b).start()

        return c

    lax.fori_loop(0, (gnb + 1) // 2, pair_body, 0)

    for k in range(SPW):
        b = k % 2
        ws = jnp.minimum((s_bnds[k] // 8) * 8, N_NODES - WR)
        hend = jnp.minimum(((s_bnds[k] + GRAN - 1) // GRAN) * GRAN,
                           s_bnds[k + 1])
        wdesc(k, b).wait()

        for h in range(HALF):
            scol = h * (D // HALF)
            carry0 = tuple(
                ostage[k, pl.ds(scol + j * L, L)] for j in range(CV)
            ) + tuple(
                ostage[k, pl.ds(D + scol + j * L, L)] for j in range(CV)
            )

            def wrow_body(r, carry):
                ro = r - ws
                fs = [wbuf[b, ro, pl.ds(scol + j * L, L)]
                      for j in range(CV)]
                sums = tuple(s + f for s, f in zip(carry[:CV], fs))
                maxs = tuple(jnp.maximum(m, f)
                             for m, f in zip(carry[CV:], fs))
                return sums + maxs

            carry = lax.fori_loop(s_bnds[k], hend, wrow_body, carry0)
            for j in range(CV):
                ostage[k, pl.ds(scol + j * L, L)] = carry[j]
                ostage[k, pl.ds(D + scol + j * L, L)] = carry[CV + j]

        if k + 2 < SPW:
            wdesc(k + 2, b).start()

    pltpu.sync_copy(ostage, out_hbm.at[pl.ds(sbase, SPW)])


def kernel(feat, segment_ids):
    lo = _tc_group_partials(feat, segment_ids)
    mesh = plsc.VectorSubcoreMesh(core_axis_name="c", subcore_axis_name="s")
    f = pl.kernel(
        _comb_body,
        out_type=jax.ShapeDtypeStruct((NSEG, 2 * D), jnp.float32),
        mesh=mesh,
        compiler_params=pltpu.CompilerParams(needs_layout_passes=False),
        scratch_types=[
            pltpu.VMEM((N_NODES,), jnp.int32),
            pltpu.VMEM((2, GR, 2 * D), jnp.float32),
            pltpu.VMEM((2, WR, D), jnp.float32),
            pltpu.VMEM((SPW, 2 * D), jnp.float32),
            pltpu.SemaphoreType.DMA,
            pltpu.SemaphoreType.DMA,
            pltpu.SemaphoreType.DMA,
            pltpu.SemaphoreType.DMA,
        ],
    )
    return f(lo, feat, segment_ids)


# speedup vs baseline: 1.4686x; 1.4686x over previous
import jax
import jax.numpy as jnp
from jax import lax
from jax.experimental import pallas as pl
from jax.experimental.pallas import tpu as pltpu
from jax.experimental.pallas import tpu_sc as plsc

N_NODES = 50000
D = 512
NSEG = 256
L = 16
NC = 2
NS = 16
NW = NC * NS
SPW = NSEG // NW
R = 64
HALF = 2
CH = D // HALF
CV = CH // L


def _pool_body(feat_hbm, ids_hbm, out_hbm, idsv, sp_ids, fbuf, ostage,
               sem0, sem1):
    wid = lax.axis_index("s") * NC + lax.axis_index("c")
    sbase = wid * SPW

    @pl.when(lax.axis_index("s") == 0)
    def _():
        pltpu.sync_copy(ids_hbm, sp_ids)

    plsc.subcore_barrier()
    pltpu.sync_copy(sp_ids, idsv)

    targets = sbase + lax.iota(jnp.int32, L)
    pos = jnp.zeros((L,), jnp.int32)
    step = 32768
    while step >= 1:
        npos = pos + step
        idx = jnp.minimum(npos - 1, N_NODES - 1)
        vals = plsc.load_gather(idsv, [idx])
        ok = (npos <= N_NODES) & (vals < targets)
        pos = jnp.where(ok, npos, pos)
        step //= 2

    s_bnds = [pos[k] for k in range(SPW + 1)]
    row_lo = s_bnds[0]
    row_hi = s_bnds[SPW]

    g_lo = row_lo // R
    nb = jnp.where(row_hi > row_lo, (row_hi + R - 1) // R - g_lo, 0)

    def bstart_of(t):
        return jnp.minimum((g_lo + t) * R, N_NODES - R)

    def copy_desc(t, b):
        buf = fbuf.at[b]
        sem = sem0 if b == 0 else sem1
        return pltpu.make_async_copy(
            feat_hbm.at[pl.ds(bstart_of(t), R)], buf, sem)

    @pl.when(nb > 0)
    def _():
        copy_desc(0, 0).start()

    @pl.when(nb > 1)
    def _():
        copy_desc(1, 1).start()

    zeros = jnp.zeros((L,), jnp.float32)
    ninf = jnp.full((L,), -jnp.inf, jnp.float32)

    def init_body(k, c):
        for j in range(D // L):
            ostage[k, pl.ds(j * L, L)] = zeros
            ostage[k, pl.ds(D + j * L, L)] = ninf
        return c

    lax.fori_loop(0, SPW, init_body, 0)

    def process(t, b):
        g = (g_lo + t) * R
        proc_lo = jnp.maximum(row_lo, g)
        proc_hi = jnp.minimum(row_hi, g + R)
        bstart = bstart_of(t)
        buf = fbuf.at[b]

        for k in range(SPW):
            a = jnp.maximum(s_bnds[k], proc_lo)
            e = jnp.minimum(s_bnds[k + 1], proc_hi)

            @pl.when(e > a)
            def _():
                for h in range(HALF):
                    scol = h * CH
                    mcol = D + h * CH
                    carry0 = tuple(
                        ostage[k, pl.ds(scol + j * L, L)] for j in range(CV)
                    ) + tuple(
                        ostage[k, pl.ds(mcol + j * L, L)] for j in range(CV)
                    )

                    def row_body(r, carry):
                        ro = r - bstart
                        fs = [buf[ro, pl.ds(scol + j * L, L)]
                              for j in range(CV)]
                        sums = tuple(s + f for s, f in zip(carry[:CV], fs))
                        maxs = tuple(jnp.maximum(m, f)
                                     for m, f in zip(carry[CV:], fs))
                        return sums + maxs

                    carry = lax.fori_loop(a, e, row_body, carry0)
                    for j in range(CV):
                        ostage[k, pl.ds(scol + j * L, L)] = carry[j]
                        ostage[k, pl.ds(mcol + j * L, L)] = carry[CV + j]

    def pair_body(u, c):
        for b in range(2):
            t = u * 2 + b

            @pl.when(t < nb)
            def _():
                copy_desc(t, b).wait()
                process(t, b)

                @pl.when(t + 2 < nb)
                def _():
                    copy_desc(t + 2, b).start()

        return c

    lax.fori_loop(0, (nb + 1) // 2, pair_body, 0)

    pltpu.sync_copy(ostage, out_hbm.at[pl.ds(sbase, SPW)])


def kernel(feat, segment_ids):
    mesh = plsc.VectorSubcoreMesh(core_axis_name="c", subcore_axis_name="s")
    f = pl.kernel(
        _pool_body,
        out_type=jax.ShapeDtypeStruct((NSEG, 2 * D), jnp.float32),
        mesh=mesh,
        compiler_params=pltpu.CompilerParams(needs_layout_passes=False),
        scratch_types=[
            pltpu.VMEM((N_NODES,), jnp.int32),
            pltpu.VMEM_SHARED((N_NODES,), jnp.int32),
            pltpu.VMEM((2, R, D), jnp.float32),
            pltpu.VMEM((SPW, 2 * D), jnp.float32),
            pltpu.SemaphoreType.DMA,
            pltpu.SemaphoreType.DMA,
        ],
    )
    return f(feat, segment_ids)
